# baseline (speedup 1.0000x reference)
import functools

import jax
import jax.numpy as jnp
from jax import lax
from jax.experimental import pallas as pl
from jax.experimental.pallas import tpu as pltpu
from jax.experimental.pallas import tpu_sc as plsc

_L = 16
_G = 8
_NS = 4


@functools.lru_cache(maxsize=None)
def _make_sc_sel(C):
    n_chunks = C // _L
    mesh = plsc.VectorSubcoreMesh(core_axis_name="c", subcore_axis_name="s")

    @functools.partial(
        pl.kernel,
        mesh=mesh,
        compiler_params=pltpu.CompilerParams(needs_layout_passes=False),
        out_type=jax.ShapeDtypeStruct((C,), jnp.int32),
        scratch_types=[
            pltpu.VMEM((C,), jnp.float32),
            pltpu.VMEM((C,), jnp.int32),
        ],
    )
    def k(indexes_hbm, sel_hbm, idxs_v, sel_v):
        wid = lax.axis_index("s") * 2 + lax.axis_index("c")
        pltpu.sync_copy(indexes_hbm, idxs_v)

        iota = lax.iota(jnp.int32, _L)
        one = jnp.int32(1)
        zero = jnp.int32(0)

        acc = jnp.zeros((_L,), jnp.int32)
        for c in range(n_chunks):
            v = idxs_v[pl.ds(c * _L, _L)]
            acc = acc + jnp.where(v != 0.0, one, zero)
        total_nz = zero
        for j in range(_L):
            total_nz = total_nz + acc[j]

        nz = zero
        z = zero
        for c in range(n_chunks):
            v = idxs_v[pl.ds(c * _L, _L)]
            posvec = jnp.zeros((_L,), jnp.int32)
            for j in range(_L):
                mj = v[j] != 0.0
                pos_j = jnp.where(mj, nz, total_nz + z)
                posvec = jnp.where(iota == j, pos_j, posvec)
                nz = nz + jnp.where(mj, one, zero)
                z = z + jnp.where(mj, zero, one)
            plsc.store_scatter(sel_v, [posvec], iota + (c * _L))

        @pl.when(wid == 0)
        def _():
            pltpu.sync_copy(sel_v, sel_hbm)

    return k


@functools.lru_cache(maxsize=None)
def _make_tc_gather(B, C, H, W):
    assert C % _G == 0
    npb = C // _G

    def body(sel_smem, in_ref, out_ref, *scratch):
        bufs = scratch[:_NS]
        gsems = scratch[_NS:2 * _NS]
        psems = scratch[2 * _NS:]
        b = pl.program_id(0)
        writes = [None] * npb

        def start_gathers(g):
            c0 = g * _G
            s = [sel_smem[c0 + j] for j in range(_G)]
            consec = s[1] == s[0] + 1
            for j in range(2, _G):
                consec = jnp.logical_and(consec, s[j] == s[j - 1] + 1)

            @pl.when(consec)
            def _():
                pltpu.make_async_copy(
                    in_ref.at[b, pl.ds(s[0], _G)],
                    bufs[g % _NS],
                    gsems[g % _NS],
                ).start()

            @pl.when(jnp.logical_not(consec))
            def _():
                for j in range(_G):
                    pltpu.make_async_copy(
                        in_ref.at[b, pl.ds(s[j], 1)],
                        bufs[g % _NS].at[pl.ds(j, 1)],
                        gsems[g % _NS],
                    ).start()

        def wait_gathers(g):
            pltpu.make_async_copy(
                in_ref.at[b, pl.ds(0, _G)], bufs[g % _NS], gsems[g % _NS]
            ).wait()

        def start_write(g):
            cp = pltpu.make_async_copy(
                bufs[g % _NS],
                out_ref.at[b, pl.ds(g * _G, _G)],
                psems[g % _NS],
            )
            cp.start()
            return cp

        for g in range(npb):
            if g >= _NS:
                writes[g - _NS].wait()
            start_gathers(g)
            if g >= 1:
                wait_gathers(g - 1)
                writes[g - 1] = start_write(g - 1)
        wait_gathers(npb - 1)
        writes[npb - 1] = start_write(npb - 1)
        for g in range(npb - _NS, npb):
            writes[g].wait()

    grid_spec = pltpu.PrefetchScalarGridSpec(
        num_scalar_prefetch=1,
        grid=(B,),
        in_specs=[pl.BlockSpec(memory_space=pl.ANY)],
        out_specs=pl.BlockSpec(memory_space=pl.ANY),
        scratch_shapes=(
            [pltpu.VMEM((_G, H, W), jnp.float32)] * _NS
            + [pltpu.SemaphoreType.DMA] * (2 * _NS)
        ),
    )
    return pl.pallas_call(
        body,
        grid_spec=grid_spec,
        out_shape=jax.ShapeDtypeStruct((B, C, H, W), jnp.float32),
    )


def kernel(input_tensor, indexes):
    B, C, H, W = input_tensor.shape
    sel = _make_sc_sel(C)(indexes)
    return _make_tc_gather(B, C, H, W)(sel, input_tensor)


# speedup vs baseline: 1.9650x; 1.9650x over previous
import functools

import jax
import jax.numpy as jnp
from jax import lax
from jax.experimental import pallas as pl
from jax.experimental.pallas import tpu as pltpu
from jax.experimental.pallas import tpu_sc as plsc

_L = 16
_G = 4
_NS = 4


@functools.lru_cache(maxsize=None)
def _make_sc_permute(B, C, H, W):
    info = plsc.get_sparse_core_info()
    NC, NS = info.num_cores, info.num_subcores
    NW = NC * NS
    assert C % _L == 0 and B % NW == 0
    bpt = B // NW
    n_chunks = C // _L
    mesh = plsc.VectorSubcoreMesh(core_axis_name="c", subcore_axis_name="s")

    @functools.partial(
        pl.kernel,
        mesh=mesh,
        compiler_params=pltpu.CompilerParams(
            needs_layout_passes=False, use_tc_tiling_on_sc=True
        ),
        out_type=jax.ShapeDtypeStruct((B * C, H, W), jnp.float32),
        scratch_types=[
            pltpu.VMEM((C,), jnp.float32),
            pltpu.VMEM((C,), jnp.int32),
        ] + [pltpu.VMEM((_G, H, W), jnp.float32)] * _NS
          + [pltpu.SemaphoreType.DMA] * (2 * _NS),
    )
    def k(indexes_hbm, in_hbm, out_hbm, idxs_v, sel_v, *bufs_and_sems):
        bufs = bufs_and_sems[:_NS]
        gsems = bufs_and_sems[_NS:2 * _NS]
        psems = bufs_and_sems[2 * _NS:]
        wid = lax.axis_index("s") * NC + lax.axis_index("c")
        pltpu.sync_copy(indexes_hbm, idxs_v)

        iota = lax.iota(jnp.int32, _L)
        one = jnp.int32(1)
        zero = jnp.int32(0)

        acc = jnp.zeros((_L,), jnp.int32)
        for c in range(n_chunks):
            v = idxs_v[pl.ds(c * _L, _L)]
            acc = acc + jnp.where(v != 0.0, one, zero)
        total_nz = zero
        for j in range(_L):
            total_nz = total_nz + acc[j]

        nz = zero
        z = zero
        for c in range(n_chunks):
            v = idxs_v[pl.ds(c * _L, _L)]
            posvec = jnp.zeros((_L,), jnp.int32)
            for j in range(_L):
                mj = v[j] != 0.0
                pos_j = jnp.where(mj, nz, total_nz + z)
                posvec = jnp.where(iota == j, pos_j, posvec)
                nz = nz + jnp.where(mj, one, zero)
                z = z + jnp.where(mj, zero, one)
            plsc.store_scatter(sel_v, [posvec], iota + (c * _L))

        out_base = wid * bpt * C
        ngroups = bpt * C // _G
        writes = [None] * ngroups

        def start_gathers(g):
            b, off = divmod(g * _G, C)
            row0 = (wid * bpt + b) * C
            chunk, lane0 = divmod(off, _L)
            vec = sel_v[pl.ds(chunk * _L, _L)] + row0
            s = [vec[lane0 + j] for j in range(_G)]
            consec = s[1] == s[0] + 1
            for j in range(2, _G):
                consec = jnp.logical_and(consec, s[j] == s[j - 1] + 1)

            @pl.when(consec)
            def _():
                pltpu.async_copy(
                    in_hbm.at[pl.ds(s[0], _G)], bufs[g % _NS], gsems[g % _NS]
                )

            @pl.when(jnp.logical_not(consec))
            def _():
                for j in range(_G):
                    pltpu.async_copy(
                        in_hbm.at[pl.ds(s[j], 1)],
                        bufs[g % _NS].at[pl.ds(j, 1)],
                        gsems[g % _NS],
                    )

        def wait_gathers(g):
            pltpu.make_async_copy(
                in_hbm.at[pl.ds(0, _G)], bufs[g % _NS], gsems[g % _NS]
            ).wait()

        def start_write(g):
            return pltpu.async_copy(
                bufs[g % _NS],
                out_hbm.at[pl.ds(out_base + g * _G, _G)],
                psems[g % _NS],
            )

        for g in range(ngroups):
            if g >= _NS:
                writes[g - _NS].wait()
            start_gathers(g)
            if g >= 1:
                wait_gathers(g - 1)
                writes[g - 1] = start_write(g - 1)
        wait_gathers(ngroups - 1)
        writes[ngroups - 1] = start_write(ngroups - 1)
        for g in range(ngroups - _NS, ngroups):
            writes[g].wait()

    return k


def kernel(input_tensor, indexes):
    B, C, H, W = input_tensor.shape
    flat = input_tensor.reshape(B * C, H, W)
    out = _make_sc_permute(B, C, H, W)(indexes, flat)
    return out.reshape(B, C, H, W)
